# baseline (speedup 1.0000x reference)
import functools

import jax
import jax.numpy as jnp
from jax import lax
from jax.experimental import pallas as pl
from jax.experimental.pallas import tpu as pltpu
from jax.experimental.pallas import tpu_sc as plsc

F_TOTAL = 4
N_ROWS = 100000
DIM = 64
BATCH = 16384
NC, NS = 2, 16
NW = NC * NS
BPW = BATCH // NW
R = 64
NCH = BPW // R
LANE = 16
DC = DIM // LANE


def _sc_body(feats_hbm, idx_i_hbm, idx_j_hbm, par_hbm, out_hbm,
             par_v, tmp_v, idx_v, gbuf, sq_v, out_v, sem):
    wid = lax.axis_index("s") * NC + lax.axis_index("c")
    base = wid * BPW

    pltpu.sync_copy(par_hbm, par_v)
    lanes = lax.iota(jnp.int32, LANE)
    zeros = jnp.zeros((LANE,), jnp.int32)

    def bcast_lane(v, l):
        idx = jnp.full((LANE, 1), l, jnp.int32)
        return lax.gather(
            v, idx,
            lax.GatherDimensionNumbers(offset_dims=(),
                                       collapsed_slice_dims=(0,),
                                       start_index_map=(0,)),
            (1,), mode=lax.GatherScatterMode.PROMISE_IN_BOUNDS)

    fw = par_v[pl.ds(0, LANE)]
    m = bcast_lane(fw, 0)
    for l in range(1, F_TOTAL):
        m = jnp.maximum(m, bcast_lane(fw, l))
    e = jnp.exp(fw - m)
    s = bcast_lane(e, 0)
    for l in range(1, F_TOTAL):
        s = s + bcast_lane(e, l)
    w = [bcast_lane(e, f) / s for f in range(F_TOTAL)]
    b_vec = par_v[pl.ds(LANE, LANE)]
    k_vec = par_v[pl.ds(2 * LANE, LANE)]

    pltpu.sync_copy(idx_i_hbm.at[pl.ds(base, BPW)], tmp_v)

    def idx_i_body(v, carry):
        sl = pl.ds(v * LANE, LANE)
        t = tmp_v[sl]
        for f in range(F_TOTAL):
            idx_v[f, sl] = t + f * N_ROWS
        return carry

    lax.fori_loop(0, BPW // LANE, idx_i_body, 0)
    pltpu.sync_copy(idx_j_hbm.at[pl.ds(base, BPW)], tmp_v)

    def idx_j_body(v, carry):
        sl = pl.ds(v * LANE, LANE)
        t = tmp_v[sl]
        for f in range(F_TOTAL):
            idx_v[F_TOTAL + f, sl] = t + f * N_ROWS
        return carry

    lax.fori_loop(0, BPW // LANE, idx_j_body, 0)

    def chunk_body(k, carry):
        copies = []
        for side in range(2 * F_TOTAL):
            cp = pltpu.make_async_copy(
                feats_hbm.at[idx_v.at[side, pl.ds(k * R, R)]],
                gbuf.at[side], sem)
            cp.start()
            copies.append(cp)
        for cp in copies:
            cp.wait()

        def blk_body(blk, carry2):
            for rl in range(LANE):
                r = blk * LANE + rl
                sq = None
                for c in range(DC):
                    cs = pl.ds(c * LANE, LANE)
                    a = (gbuf[0, r, cs] - gbuf[F_TOTAL, r, cs]) * w[0]
                    for f in range(1, F_TOTAL):
                        a += (gbuf[f, r, cs] - gbuf[F_TOTAL + f, r, cs]) * w[f]
                    sq = a * a if sq is None else sq + a * a
                sq_v[pl.ds(rl * LANE, LANE)] = sq
            o = plsc.load_gather(sq_v, [lanes * LANE])
            for l in range(1, LANE):
                o += plsc.load_gather(sq_v, [lanes * LANE + l])
            logit = b_vec - k_vec * o
            eneg = jnp.exp(-jnp.abs(logit))
            inv = 1.0 / (1.0 + eneg)
            res = jnp.where(logit >= 0.0, inv, eneg * inv)
            out_v[pl.ds(k * R + blk * LANE, LANE)] = res
            return carry2

        lax.fori_loop(0, R // LANE, blk_body, 0)
        return carry

    lax.fori_loop(0, NCH, chunk_body, 0)
    pltpu.sync_copy(out_v, out_hbm.at[pl.ds(base, BPW)])


_sc_kernel = functools.partial(
    pl.kernel,
    out_type=jax.ShapeDtypeStruct((BATCH,), jnp.float32),
    mesh=plsc.VectorSubcoreMesh(core_axis_name="c", subcore_axis_name="s"),
    compiler_params=pltpu.CompilerParams(needs_layout_passes=False,
                                         use_tc_tiling_on_sc=False),
    scratch_types=[
        pltpu.VMEM((3 * LANE,), jnp.float32),
        pltpu.VMEM((BPW,), jnp.int32),
        pltpu.VMEM((2 * F_TOTAL, BPW), jnp.int32),
        pltpu.VMEM((2 * F_TOTAL, R, DIM), jnp.float32),
        pltpu.VMEM((LANE * LANE,), jnp.float32),
        pltpu.VMEM((BPW,), jnp.float32),
        pltpu.SemaphoreType.DMA,
    ],
)(_sc_body)


@jax.jit
def kernel(features, feature_weights, intercept, slope, idx_i, idx_j):
    feats2d = features.reshape(F_TOTAL * N_ROWS, DIM)
    fw = jnp.full((LANE,), -1e30, dtype=jnp.float32)
    fw = fw.at[:F_TOTAL].set(feature_weights.reshape(-1).astype(jnp.float32))
    par = jnp.concatenate([
        fw,
        jnp.full((LANE,), intercept, dtype=jnp.float32),
        jnp.full((LANE,), slope, dtype=jnp.float32),
    ])
    return _sc_kernel(feats2d, idx_i, idx_j, par)
